# Initial kernel scaffold; baseline (speedup 1.0000x reference)
import jax
import jax.numpy as jnp
from jax.experimental import pallas as pl


def kernel(x, edge_index, W1, b1, W2, b2):
    raise NotImplementedError("write your pallas kernel here")


# speedup vs baseline: 13.6224x; 13.6224x over previous
import jax
import jax.numpy as jnp
from jax import lax
from jax.experimental import pallas as pl
from jax.experimental.pallas import tpu as pltpu
from jax.experimental.pallas import tpu_sc as plsc

N_NODES = 10000
N_EDGES = 320000
D = 128

NC, NS, LANES = 2, 16, 16
NW = NC * NS
NPAD = 10240
ROWS_PER_TILE = NPAD // NS
CH = 128
K = 79
EPW = K * CH
EPAD = NW * EPW
BM = 512

_MESH = plsc.VectorSubcoreMesh(
    core_axis_name="c", subcore_axis_name="s", num_cores=NC, num_subcores=NS
)


def _deg_body(dst_hbm, out_hbm, dst_loc, deg_loc, red_v, sum_v, part_sh):
    c = lax.axis_index("c")
    s = lax.axis_index("s")
    w = s * NC + c

    zero16 = jnp.zeros((LANES,), jnp.float32)

    def zbody(i, carry):
        deg_loc[pl.ds(i * LANES, LANES)] = zero16
        return carry

    lax.fori_loop(0, NPAD // LANES, zbody, 0)

    pltpu.sync_copy(dst_hbm.at[w], dst_loc)

    ones16 = jnp.ones((LANES,), jnp.float32)

    def hbody(k, carry):
        idx = dst_loc[pl.ds(k * LANES, LANES)]
        plsc.addupdate_scatter(deg_loc, [idx], ones16)
        return carry

    lax.fori_loop(0, EPW // LANES, hbody, 0)

    pltpu.sync_copy(deg_loc, part_sh.at[s])
    plsc.subcore_barrier()

    pltpu.sync_copy(part_sh.at[:, pl.ds(s * ROWS_PER_TILE, ROWS_PER_TILE)], red_v)

    def rbody(j, carry):
        acc = jnp.zeros((LANES,), jnp.float32)
        for r in range(NS):
            acc = acc + red_v[r, pl.ds(j * LANES, LANES)]
        sum_v[pl.ds(j * LANES, LANES)] = acc
        return carry

    lax.fori_loop(0, ROWS_PER_TILE // LANES, rbody, 0)

    pltpu.sync_copy(sum_v, out_hbm.at[c, pl.ds(s * ROWS_PER_TILE, ROWS_PER_TILE)])


_deg_call = pl.kernel(
    _deg_body,
    out_type=jax.ShapeDtypeStruct((NC, NPAD), jnp.float32),
    mesh=_MESH,
    scratch_types=[
        pltpu.VMEM((EPW,), jnp.int32),
        pltpu.VMEM((NPAD,), jnp.float32),
        pltpu.VMEM((NS, ROWS_PER_TILE), jnp.float32),
        pltpu.VMEM((ROWS_PER_TILE,), jnp.float32),
        pltpu.VMEM_SHARED((NS, NPAD), jnp.float32),
    ],
    compiler_params=pltpu.CompilerParams(needs_layout_passes=False),
)


def _agg_body(h_hbm, src_hbm, dst_hbm, out_hbm, idx_src, idx_dst, rows, sem, acc_sh):
    c = lax.axis_index("c")
    s = lax.axis_index("s")
    w = s * NC + c

    pltpu.sync_copy(src_hbm.at[w], idx_src)
    pltpu.sync_copy(dst_hbm.at[w], idx_dst)

    r0 = s * ROWS_PER_TILE
    for j in range(ROWS_PER_TILE // CH):
        pltpu.sync_copy(h_hbm.at[pl.ds(r0 + j * CH, CH)], rows)
        pltpu.sync_copy(rows, acc_sh.at[pl.ds(r0 + j * CH, CH)])
    plsc.subcore_barrier()

    def ebody(k, carry):
        pltpu.async_copy(h_hbm.at[idx_src.at[k]], rows, sem).wait()
        pltpu.sync_copy(rows, acc_sh.at[idx_dst.at[k]], add=True)
        return carry

    lax.fori_loop(0, K, ebody, 0)

    plsc.subcore_barrier()
    for j in range(ROWS_PER_TILE // CH):
        pltpu.sync_copy(acc_sh.at[pl.ds(r0 + j * CH, CH)], rows)
        pltpu.sync_copy(rows, out_hbm.at[c, pl.ds(r0 + j * CH, CH)])


_agg_call = pl.kernel(
    _agg_body,
    out_type=jax.ShapeDtypeStruct((NC, NPAD, D), jnp.float32),
    mesh=_MESH,
    scratch_types=[
        pltpu.VMEM((K, CH), jnp.int32),
        pltpu.VMEM((K, CH), jnp.int32),
        pltpu.VMEM((CH, D), jnp.float32),
        pltpu.SemaphoreType.DMA,
        pltpu.VMEM_SHARED((NPAD, D), jnp.float32),
    ],
)


def _dinv(dp_ref):
    deg = dp_ref[0] + dp_ref[1] + 1.0
    return lax.rsqrt(jnp.maximum(deg, 1e-12))


def _mm1_body(dp_ref, x_ref, w_ref, o_ref):
    dinv = _dinv(dp_ref)
    h = jnp.dot(x_ref[...], w_ref[...], preferred_element_type=jnp.float32)
    o_ref[...] = h * dinv


def _mid_body(dp_ref, p_ref, h1_ref, b1_ref, w2_ref, o_ref):
    dinv = _dinv(dp_ref)
    agg = p_ref[0] + p_ref[1] - h1_ref[...]
    t = jnp.maximum(agg * dinv + b1_ref[...], 0.0)
    h2 = jnp.dot(t, w2_ref[...], preferred_element_type=jnp.float32)
    o_ref[...] = h2 * dinv


def _fin_body(dp_ref, q_ref, h2_ref, b2_ref, o_ref):
    dinv = _dinv(dp_ref)
    o_ref[...] = (q_ref[0] + q_ref[1] - h2_ref[...]) * dinv + b2_ref[...]


_G = NPAD // BM

_mm1_call = pl.pallas_call(
    _mm1_body,
    grid=(_G,),
    in_specs=[
        pl.BlockSpec((NC, BM, 1), lambda i: (0, i, 0)),
        pl.BlockSpec((BM, D), lambda i: (i, 0)),
        pl.BlockSpec((D, D), lambda i: (0, 0)),
    ],
    out_specs=pl.BlockSpec((BM, D), lambda i: (i, 0)),
    out_shape=jax.ShapeDtypeStruct((NPAD, D), jnp.float32),
)

_mid_call = pl.pallas_call(
    _mid_body,
    grid=(_G,),
    in_specs=[
        pl.BlockSpec((NC, BM, 1), lambda i: (0, i, 0)),
        pl.BlockSpec((NC, BM, D), lambda i: (0, i, 0)),
        pl.BlockSpec((BM, D), lambda i: (i, 0)),
        pl.BlockSpec((1, D), lambda i: (0, 0)),
        pl.BlockSpec((D, D), lambda i: (0, 0)),
    ],
    out_specs=pl.BlockSpec((BM, D), lambda i: (i, 0)),
    out_shape=jax.ShapeDtypeStruct((NPAD, D), jnp.float32),
)

_fin_call = pl.pallas_call(
    _fin_body,
    grid=(_G,),
    in_specs=[
        pl.BlockSpec((NC, BM, 1), lambda i: (0, i, 0)),
        pl.BlockSpec((NC, BM, D), lambda i: (0, i, 0)),
        pl.BlockSpec((BM, D), lambda i: (i, 0)),
        pl.BlockSpec((1, D), lambda i: (0, 0)),
    ],
    out_specs=pl.BlockSpec((BM, D), lambda i: (i, 0)),
    out_shape=jax.ShapeDtypeStruct((NPAD, D), jnp.float32),
)


def kernel(x, edge_index, W1, b1, W2, b2):
    src = edge_index[0].astype(jnp.int32)
    dst = edge_index[1].astype(jnp.int32)
    pad = jnp.full((EPAD - N_EDGES,), N_NODES, jnp.int32)
    src3 = jnp.concatenate([src, pad]).reshape(NW, K, CH)
    dst3 = jnp.concatenate([dst, pad]).reshape(NW, K, CH)
    dst2 = dst3.reshape(NW, EPW)
    xp = jnp.zeros((NPAD, D), jnp.float32).at[:N_NODES].set(x)

    deg_parts = _deg_call(dst2)
    dp3 = deg_parts.reshape(NC, NPAD, 1)
    h1 = _mm1_call(dp3, xp, W1)
    p = _agg_call(h1, src3, dst3)
    h2 = _mid_call(dp3, p, h1, b1.reshape(1, D), W2)
    q = _agg_call(h2, src3, dst3)
    outp = _fin_call(dp3, q, h2, b2.reshape(1, D))
    return outp[:N_NODES]
